# baseline (speedup 1.0000x reference)
import functools

import jax
import jax.numpy as jnp
from jax import lax
from jax.experimental import pallas as pl
from jax.experimental.pallas import tpu as pltpu
from jax.experimental.pallas import tpu_sc as plsc

_OUT_D = 64
_MAX_POS = 1024
_CHUNK = 128


def _pos_freqs(nt: int) -> jnp.ndarray:
    dim = _OUT_D
    inv = 1.0 / (10000.0 ** (jnp.arange(0, dim, 2)[: dim // 2].astype(jnp.float32) / dim))
    pos = jnp.minimum(jnp.arange(nt, dtype=jnp.float32), float(_MAX_POS - 1))
    f = pos[:, None] * inv[None, :]
    return jnp.concatenate([jnp.cos(f), jnp.sin(f)], axis=-1)


def kernel(text, text_embed_weight):
    b, nt = text.shape
    d = text_embed_weight.shape[1]
    total = b * nt

    info = plsc.get_sparse_core_info()
    nc, ns = info.num_cores, info.num_subcores
    nw = nc * ns
    per_w = total // nw
    nchunk = per_w // _CHUNK
    assert d == _OUT_D
    assert per_w * nw == total
    assert nchunk * _CHUNK == per_w
    assert per_w % nt == 0
    assert nchunk % 3 == 0 or nchunk % 3 == 2

    fq2 = jnp.concatenate([_pos_freqs(nt)] * 2, axis=0)

    idx = text.reshape(nw, nchunk, _CHUNK).astype(jnp.int32)

    mesh = plsc.VectorSubcoreMesh(core_axis_name="c", subcore_axis_name="s")

    @functools.partial(
        pl.kernel,
        mesh=mesh,
        compiler_params=pltpu.CompilerParams(use_tc_tiling_on_sc=False),
        out_type=jax.ShapeDtypeStruct((total, d), jnp.float32),
        scratch_types=[
            pltpu.VMEM((nchunk, _CHUNK), jnp.int32),
            pltpu.VMEM((2 * nt, d), jnp.float32),
            pltpu.VMEM((_CHUNK, d), jnp.float32),
            pltpu.VMEM((_CHUNK, d), jnp.float32),
            pltpu.VMEM((_CHUNK, d), jnp.float32),
            pltpu.SemaphoreType.DMA,
            pltpu.SemaphoreType.DMA,
            pltpu.SemaphoreType.DMA,
            pltpu.SemaphoreType.DMA,
            pltpu.SemaphoreType.DMA,
            pltpu.SemaphoreType.DMA,
        ],
    )
    def emb_kernel(w_hbm, idx_hbm, fq_hbm, out_hbm, idx_v, fq_v,
                   buf0, buf1, buf2, sem0, sem1, sem2, so0, so1, so2):
        wid = lax.axis_index("s") * nc + lax.axis_index("c")
        base = wid * per_w
        pltpu.sync_copy(idx_hbm.at[wid], idx_v)
        pltpu.sync_copy(fq_hbm, fq_v)

        bufs = (buf0, buf1, buf2)
        sems = (sem0, sem1, sem2)
        ssems = (so0, so1, so2)

        def bump(row):
            for c in range(_CHUNK // 16):
                sl = pl.ds(c * 16, 16)
                idx_v[row, sl] = idx_v[row, sl] + 1

        def start_gather(k, bf):
            pltpu.async_copy(w_hbm.at[idx_v.at[k]], bufs[bf], sems[bf])

        def wait_gather(bf):
            pltpu.make_async_copy(w_hbm.at[idx_v.at[0]], bufs[bf],
                                  sems[bf]).wait()

        def wait_store(bf):
            pltpu.make_async_copy(w_hbm.at[idx_v.at[0]], bufs[bf],
                                  ssems[bf]).wait()

        bump(0)
        start_gather(0, 0)
        bump(1)
        start_gather(1, 1)

        def chunk_body(k, bf):
            @pl.when(k + 2 < nchunk)
            def _():
                @pl.when(k >= 1)
                def _():
                    wait_store((bf + 2) % 3)

                bump(k + 2)
                start_gather(k + 2, (bf + 2) % 3)

            wait_gather(bf)
            buf = bufs[bf]
            t0 = lax.rem(k * _CHUNK, nt)

            def row_body(r, carry):
                t = t0 + r
                for c in range(d // 16):
                    sl = pl.ds(c * 16, 16)
                    buf[r, sl] = buf[r, sl] + fq_v[t, sl]
                return carry

            lax.fori_loop(0, _CHUNK, row_body, 0, unroll=4)
            pltpu.async_copy(buf, out_hbm.at[pl.ds(base + k * _CHUNK, _CHUNK)],
                             ssems[bf])

        def outer(i, carry):
            chunk_body(3 * i, 0)
            chunk_body(3 * i + 1, 1)
            chunk_body(3 * i + 2, 2)
            return carry

        lax.fori_loop(0, nchunk // 3, outer, 0)
        if nchunk % 3 == 2:
            chunk_body(nchunk - 2, 0)
            chunk_body(nchunk - 1, 1)
        wait_store(0)
        wait_store(1)
        wait_store(2)

    out = emb_kernel(text_embed_weight, idx, fq2)
    return out.reshape(b, nt, d)
